# Initial kernel scaffold; baseline (speedup 1.0000x reference)
import functools

import jax
import jax.numpy as jnp
from jax import lax
from jax.experimental import pallas as pl
from jax.experimental.pallas import tpu as pltpu
from jax.experimental.pallas import tpu_sc as plsc

T, D, H, HS, E, DFF = 2048, 768, 12, 64, 8, 3072
BQ = 256
BT = 128
NBP = T // BT + E
T_PAD = NBP * BT

SC_NC, SC_NS = 2, 16
SC_NW = SC_NC * SC_NS
RW = T // SC_NW


def _ln_block(x, g, b):
    m = jnp.mean(x, axis=-1, keepdims=True)
    v = jnp.mean((x - m) ** 2, axis=-1, keepdims=True)
    return (x - m) * jax.lax.rsqrt(v + 1e-5) * g + b


def _qkv_kernel(x_ref, g_ref, b_ref, w_ref, o_ref):
    x1 = _ln_block(x_ref[...], g_ref[...], b_ref[...])
    o_ref[...] = lax.dot_general(
        x1, w_ref[...], (((1,), (1,)), ((), ())),
        preferred_element_type=jnp.float32)


def _qkv_stage(xf, ln1_g, ln1_b, wqkv):
    return pl.pallas_call(
        _qkv_kernel,
        grid=(T // BQ,),
        in_specs=[
            pl.BlockSpec((BQ, D), lambda i: (i, 0)),
            pl.BlockSpec((1, D), lambda i: (0, 0)),
            pl.BlockSpec((1, D), lambda i: (0, 0)),
            pl.BlockSpec((3 * D, D), lambda i: (0, 0)),
        ],
        out_specs=pl.BlockSpec((BQ, 3 * D), lambda i: (i, 0)),
        out_shape=jax.ShapeDtypeStruct((T, 3 * D), jnp.float32),
    )(xf, ln1_g.reshape(1, D), ln1_b.reshape(1, D), wqkv)


def _attn_kernel(q_ref, kv_ref, o_ref):
    i = pl.program_id(0)
    scale = D ** -0.5
    rows = lax.broadcasted_iota(jnp.int32, (BQ, T), 0) + i * BQ
    cols = lax.broadcasted_iota(jnp.int32, (BQ, T), 1)
    neg = jnp.float32(-1e30)
    outs = []
    for h in range(H):
        qh = q_ref[:, h * HS:(h + 1) * HS]
        kh = kv_ref[:, D + h * HS: D + (h + 1) * HS]
        vh = kv_ref[:, 2 * D + h * HS: 2 * D + (h + 1) * HS]
        s = lax.dot_general(qh, kh, (((1,), (1,)), ((), ())),
                            preferred_element_type=jnp.float32) * scale
        s = jnp.where(cols <= rows, s, neg)
        m = jnp.max(s, axis=1, keepdims=True)
        p = jnp.exp(s - m)
        p = p / jnp.sum(p, axis=1, keepdims=True)
        outs.append(jnp.dot(p, vh, preferred_element_type=jnp.float32))
    o_ref[...] = jnp.concatenate(outs, axis=1)


def _attn_stage(qkv):
    return pl.pallas_call(
        _attn_kernel,
        grid=(T // BQ,),
        in_specs=[
            pl.BlockSpec((BQ, 3 * D), lambda i: (i, 0)),
            pl.BlockSpec((T, 3 * D), lambda i: (0, 0)),
        ],
        out_specs=pl.BlockSpec((BQ, D), lambda i: (i, 0)),
        out_shape=jax.ShapeDtypeStruct((T, D), jnp.float32),
    )(qkv, qkv)


def _post_kernel(x_ref, a_ref, wp_ref, bp_ref, g2_ref, b2_ref, wg_ref,
                 xr_ref, x2_ref, sel_ref):
    att = a_ref[...]
    proj = lax.dot_general(att, wp_ref[...], (((1,), (1,)), ((), ())),
                           preferred_element_type=jnp.float32)
    xr = x_ref[...] + proj + bp_ref[...]
    x2 = _ln_block(xr, g2_ref[...], b2_ref[...])
    gate = lax.dot_general(x2, wg_ref[...], (((1,), (1,)), ((), ())),
                           preferred_element_type=jnp.float32)
    xr_ref[...] = xr
    x2_ref[...] = x2
    sel_ref[...] = jnp.argmax(gate, axis=1).astype(jnp.int32).reshape(BQ, 1)


def _post_stage(xf, att, Wp, bp, ln2_g, ln2_b, Wg):
    return pl.pallas_call(
        _post_kernel,
        grid=(T // BQ,),
        in_specs=[
            pl.BlockSpec((BQ, D), lambda i: (i, 0)),
            pl.BlockSpec((BQ, D), lambda i: (i, 0)),
            pl.BlockSpec((D, D), lambda i: (0, 0)),
            pl.BlockSpec((1, D), lambda i: (0, 0)),
            pl.BlockSpec((1, D), lambda i: (0, 0)),
            pl.BlockSpec((1, D), lambda i: (0, 0)),
            pl.BlockSpec((E, D), lambda i: (0, 0)),
        ],
        out_specs=[
            pl.BlockSpec((BQ, D), lambda i: (i, 0)),
            pl.BlockSpec((BQ, D), lambda i: (i, 0)),
            pl.BlockSpec((BQ, 1), lambda i: (i, 0)),
        ],
        out_shape=[
            jax.ShapeDtypeStruct((T, D), jnp.float32),
            jax.ShapeDtypeStruct((T, D), jnp.float32),
            jax.ShapeDtypeStruct((T, 1), jnp.int32),
        ],
    )(xf, att, Wp, bp.reshape(1, D), ln2_g.reshape(1, D),
      ln2_b.reshape(1, D), Wg)


def _route_kernel(sel_ref, pos_ref, be_ref, nu_ref):
    sel = sel_ref[...]
    eids = lax.broadcasted_iota(jnp.int32, (T, E), 1)
    onehot = (sel == eids).astype(jnp.int32)
    inc = onehot
    k = 1
    while k < T:
        shifted = jnp.concatenate(
            [jnp.zeros((k, E), jnp.int32), inc[:T - k, :]], axis=0)
        inc = inc + shifted
        k *= 2
    rank = inc - onehot
    counts = inc[T - 1:T, :]
    nblk = (counts + BT - 1) // BT
    cum = nblk
    k = 1
    while k < E:
        cum = cum + jnp.concatenate(
            [jnp.zeros((1, k), jnp.int32), cum[:, :E - k]], axis=1)
        k *= 2
    poff = BT * (cum - nblk)
    pos = jnp.sum(onehot * (poff + rank), axis=1, keepdims=True)
    pos_ref[...] = pos
    bids = lax.broadcasted_iota(jnp.int32, (NBP, E), 0)
    be = jnp.sum((jnp.broadcast_to(cum, (NBP, E)) <= bids).astype(jnp.int32),
                 axis=1, keepdims=True)
    be_ref[...] = jnp.minimum(be, E - 1)
    nu_ref[...] = jnp.broadcast_to(cum[:, E - 1:E], (8, 1))


def _route_stage(sel):
    return pl.pallas_call(
        _route_kernel,
        in_specs=[pl.BlockSpec((T, 1), lambda: (0, 0))],
        out_specs=[
            pl.BlockSpec((T, 1), lambda: (0, 0)),
            pl.BlockSpec((NBP, 1), lambda: (0, 0)),
            pl.BlockSpec((8, 1), lambda: (0, 0)),
        ],
        out_shape=[
            jax.ShapeDtypeStruct((T, 1), jnp.int32),
            jax.ShapeDtypeStruct((NBP, 1), jnp.int32),
            jax.ShapeDtypeStruct((8, 1), jnp.int32),
        ],
        grid=(),
    )(sel)


def _sc_scatter_stage(x2, xr, pos):
    mesh = plsc.VectorSubcoreMesh(core_axis_name="c", subcore_axis_name="s")

    @functools.partial(
        pl.kernel, mesh=mesh,
        out_type=[
            jax.ShapeDtypeStruct((T_PAD, D), jnp.float32),
            jax.ShapeDtypeStruct((T_PAD, D), jnp.float32),
        ],
        scratch_types=[
            pltpu.VMEM((RW,), jnp.int32),
            pltpu.VMEM((RW, D), jnp.float32),
            pltpu.VMEM((RW, D), jnp.float32),
            pltpu.SemaphoreType.DMA,
            pltpu.SemaphoreType.DMA,
        ],
    )
    def k(x2_hbm, xr_hbm, pos_hbm, x2s_hbm, xrs_hbm,
          idx_v, buf_a, buf_b, sem_a, sem_b):
        wid = lax.axis_index("s") * SC_NC + lax.axis_index("c")
        base = wid * RW
        pltpu.sync_copy(pos_hbm.at[pl.ds(base, RW)], idx_v)
        pltpu.sync_copy(x2_hbm.at[pl.ds(base, RW)], buf_a)
        pltpu.sync_copy(xr_hbm.at[pl.ds(base, RW)], buf_b)
        a = pltpu.async_copy(buf_a, x2s_hbm.at[idx_v], sem_a)
        b = pltpu.async_copy(buf_b, xrs_hbm.at[idx_v], sem_b)
        a.wait()
        b.wait()

    return k(x2, xr, pos)


def _ffn_kernel(be_ref, nu_ref, x2s_ref, xrs_ref, w1_ref, b1_ref,
                w2_ref, b2_ref, o_ref):
    i = pl.program_id(0)

    @pl.when(i < nu_ref[0])
    def _():
        x = x2s_ref[...]
        h = lax.dot_general(x, w1_ref[0], (((1,), (1,)), ((), ())),
                            preferred_element_type=jnp.float32)
        h = jnp.maximum(h + b1_ref[0], 0.0)
        y = lax.dot_general(h, w2_ref[0], (((1,), (1,)), ((), ())),
                            preferred_element_type=jnp.float32)
        o_ref[...] = y + b2_ref[0] + xrs_ref[...]


def _ffn_stage(x2s, xrs, W1, b1, W2, b2, blk_expert, nused):
    grid_spec = pltpu.PrefetchScalarGridSpec(
        num_scalar_prefetch=2,
        grid=(NBP,),
        in_specs=[
            pl.BlockSpec((BT, D), lambda i, be, nu: (i, 0)),
            pl.BlockSpec((BT, D), lambda i, be, nu: (i, 0)),
            pl.BlockSpec((1, DFF, D), lambda i, be, nu: (be[i], 0, 0)),
            pl.BlockSpec((1, 1, DFF), lambda i, be, nu: (be[i], 0, 0)),
            pl.BlockSpec((1, D, DFF), lambda i, be, nu: (be[i], 0, 0)),
            pl.BlockSpec((1, 1, D), lambda i, be, nu: (be[i], 0, 0)),
        ],
        out_specs=pl.BlockSpec((BT, D), lambda i, be, nu: (i, 0)),
    )
    return pl.pallas_call(
        _ffn_kernel,
        grid_spec=grid_spec,
        out_shape=jax.ShapeDtypeStruct((T_PAD, D), jnp.float32),
    )(blk_expert, nused, x2s, xrs, W1, b1.reshape(E, 1, DFF), W2,
      b2.reshape(E, 1, D))


def _sc_gather_stage(outs, pos):
    mesh = plsc.VectorSubcoreMesh(core_axis_name="c", subcore_axis_name="s")

    @functools.partial(
        pl.kernel, mesh=mesh,
        out_type=jax.ShapeDtypeStruct((T, D), jnp.float32),
        scratch_types=[
            pltpu.VMEM((RW,), jnp.int32),
            pltpu.VMEM((RW, D), jnp.float32),
            pltpu.SemaphoreType.DMA,
        ],
    )
    def k(outs_hbm, pos_hbm, out_hbm, idx_v, buf, sem):
        wid = lax.axis_index("s") * SC_NC + lax.axis_index("c")
        base = wid * RW
        pltpu.sync_copy(pos_hbm.at[pl.ds(base, RW)], idx_v)
        pltpu.async_copy(outs_hbm.at[idx_v], buf, sem).wait()
        pltpu.sync_copy(buf, out_hbm.at[pl.ds(base, RW)])

    return k(outs, pos)


def kernel(x, ln1_g, ln1_b, Wq, Wk, Wv, Wp, bp, ln2_g, ln2_b,
           Wg, W1, b1, W2, b2):
    Bx = x.shape[0]
    xf = x.reshape(T, D)
    wqkv = jnp.concatenate([
        Wq.reshape(H * HS, D),
        Wk.reshape(H * HS, D),
        Wv.reshape(H * HS, D),
    ], axis=0)
    qkv = _qkv_stage(xf, ln1_g, ln1_b, wqkv)
    att = _attn_stage(qkv)
    xr, x2, sel = _post_stage(xf, att, Wp, bp, ln2_g, ln2_b, Wg)
    pos2d, blk_expert, nused = _route_stage(sel)
    pos = pos2d.reshape(T)
    x2s, xrs = _sc_scatter_stage(x2, xr, pos)
    outs = _ffn_stage(x2s, xrs, W1, b1, W2, b2,
                      blk_expert.reshape(NBP), nused.reshape(8)[:1])
    out = _sc_gather_stage(outs, pos)
    return out.reshape(Bx, T, D)


# speedup vs baseline: 1.4160x; 1.4160x over previous
import functools

import jax
import jax.numpy as jnp
from jax import lax
from jax.experimental import pallas as pl
from jax.experimental.pallas import tpu as pltpu
from jax.experimental.pallas import tpu_sc as plsc

T, D, H, HS, E, DFF = 2048, 768, 12, 64, 8, 3072
BQ = 256
BT = 128
NBP = T // BT + E
T_PAD = NBP * BT

SC_NC, SC_NS = 2, 16
SC_NW = SC_NC * SC_NS
RW = T // SC_NW

DN11 = (((1,), (1,)), ((), ()))
DN00 = (((0,), (0,)), ((), ()))
DN10 = (((1,), (0,)), ((), ()))
DN01 = (((0,), (1,)), ((), ()))


def _halv(acc):
    w = acc.shape[1]
    while w > 1:
        w //= 2
        acc = acc[:, :w] + acc[:, w:]
    return acc


def _xsum(x):
    nc = x.shape[1] // 128
    a = x[:, :128]
    for i in range(1, nc):
        a = a + x[:, i * 128:(i + 1) * 128]
    s = None
    for i in range(0, 128, 8):
        g = a[:, i:i + 8]
        s = g if s is None else s + g
    return _halv(s)


def _subsum(x):
    a = x[:8]
    for i in range(1, x.shape[0] // 8):
        a = a + x[i * 8:(i + 1) * 8]
    a = a[:4] + a[4:]
    a = a[:2] + a[2:]
    return a[:1] + a[1:]


def _ln_rows(x, g, b):
    n = x.shape[1]
    m = _xsum(x) / n
    v = _xsum((x - m) ** 2) / n
    return (x - m) / jnp.sqrt(v + 1e-5) * g + b


def _qkv_kernel(x_ref, g_ref, b_ref, wqk_ref, wv_ref, qkt_ref, v_ref):
    x1 = _ln_rows(x_ref[...], g_ref[...], b_ref[...])
    qkt = lax.dot_general(wqk_ref[...], x1, DN11,
                          preferred_element_type=jnp.float32)
    qkt_ref[...] = qkt.astype(jnp.bfloat16)
    v_ref[...] = lax.dot_general(x1, wv_ref[...], DN11,
                                 preferred_element_type=jnp.float32)


def _qkv_stage(xf, ln1_g, ln1_b, wqk, wv):
    return pl.pallas_call(
        _qkv_kernel,
        grid=(T // BQ,),
        in_specs=[
            pl.BlockSpec((BQ, D), lambda i: (i, 0)),
            pl.BlockSpec((1, D), lambda i: (0, 0)),
            pl.BlockSpec((1, D), lambda i: (0, 0)),
            pl.BlockSpec((2 * D, D), lambda i: (0, 0)),
            pl.BlockSpec((D, D), lambda i: (0, 0)),
        ],
        out_specs=[
            pl.BlockSpec((2 * D, BQ), lambda i: (0, i)),
            pl.BlockSpec((BQ, D), lambda i: (i, 0)),
        ],
        out_shape=[
            jax.ShapeDtypeStruct((2 * D, T), jnp.bfloat16),
            jax.ShapeDtypeStruct((T, D), jnp.float32),
        ],
    )(xf, ln1_g.reshape(1, D), ln1_b.reshape(1, D), wqk, wv)


def _attn_kernel(qt_ref, kt_ref, v_ref, o_ref):
    i = pl.program_id(0)
    scale = jnp.float32(D ** -0.5)
    rows = lax.broadcasted_iota(jnp.int32, (BQ, T), 0) + i * BQ
    cols = lax.broadcasted_iota(jnp.int32, (BQ, T), 1)
    outs = []
    for h in range(H):
        qth = qt_ref[h * HS:(h + 1) * HS, :]
        kth = kt_ref[D + h * HS: D + (h + 1) * HS, :]
        vh = v_ref[:, h * HS:(h + 1) * HS]
        s = lax.dot_general(qth, kth, DN00,
                            preferred_element_type=jnp.float32) * scale
        s = jnp.where(cols <= rows, s, -jnp.inf)
        m = jnp.max(s, axis=1, keepdims=True)
        p = jnp.exp(s - m)
        p = p / _xsum(p)
        outs.append(lax.dot_general(vh, p, DN01,
                                    preferred_element_type=jnp.float32))
    o_ref[...] = jnp.concatenate(outs, axis=0).astype(jnp.bfloat16)


def _attn_stage(qkt, v):
    return pl.pallas_call(
        _attn_kernel,
        grid=(T // BQ,),
        in_specs=[
            pl.BlockSpec((D, BQ), lambda i: (0, i)),
            pl.BlockSpec((2 * D, T), lambda i: (0, 0)),
            pl.BlockSpec((T, D), lambda i: (0, 0)),
        ],
        out_specs=pl.BlockSpec((D, BQ), lambda i: (0, i)),
        out_shape=jax.ShapeDtypeStruct((D, T), jnp.bfloat16),
    )(qkt, qkt, v)


def _post_kernel(x_ref, at_ref, wp_ref, bp_ref, g2_ref, b2_ref, wg_ref,
                 xr_ref, x2_ref, sel_ref):
    xt = x_ref[...].T
    projt = lax.dot_general(wp_ref[...], at_ref[...], DN10,
                            preferred_element_type=jnp.float32)
    xrt = xt + (projt + bp_ref[...])
    n = xrt.shape[0]
    m = _subsum(xrt) / n
    var = _subsum((xrt - m) ** 2) / n
    x2t = (xrt - m) / jnp.sqrt(var + 1e-5) * g2_ref[...] + b2_ref[...]
    glt = lax.dot_general(wg_ref[...], x2t, DN10,
                          preferred_element_type=jnp.float32)
    mx = jnp.max(glt, axis=0, keepdims=True)
    eidx = lax.broadcasted_iota(jnp.int32, (E, BQ), 0)
    sel = jnp.min(jnp.where(glt == mx, eidx, E), axis=0, keepdims=True)
    xr_ref[...] = xrt.T
    x2_ref[...] = x2t.T
    sel_ref[...] = jnp.broadcast_to(sel, (8, BQ))


def _post_stage(xf, att, Wp, bp, ln2_g, ln2_b, Wg):
    return pl.pallas_call(
        _post_kernel,
        grid=(T // BQ,),
        in_specs=[
            pl.BlockSpec((BQ, D), lambda i: (i, 0)),
            pl.BlockSpec((D, BQ), lambda i: (0, i)),
            pl.BlockSpec((D, D), lambda i: (0, 0)),
            pl.BlockSpec((D, 1), lambda i: (0, 0)),
            pl.BlockSpec((D, 1), lambda i: (0, 0)),
            pl.BlockSpec((D, 1), lambda i: (0, 0)),
            pl.BlockSpec((E, D), lambda i: (0, 0)),
        ],
        out_specs=[
            pl.BlockSpec((BQ, D), lambda i: (i, 0)),
            pl.BlockSpec((BQ, D), lambda i: (i, 0)),
            pl.BlockSpec((8, BQ), lambda i: (0, i)),
        ],
        out_shape=[
            jax.ShapeDtypeStruct((T, D), jnp.float32),
            jax.ShapeDtypeStruct((T, D), jnp.float32),
            jax.ShapeDtypeStruct((8, T), jnp.int32),
        ],
    )(xf, att, Wp, bp.reshape(D, 1), ln2_g.reshape(D, 1),
      ln2_b.reshape(D, 1), Wg)


def _route_kernel(sel_ref, pos_ref, be_ref, nu_ref):
    selt = sel_ref[...]
    eids = lax.broadcasted_iota(jnp.int32, (E, T), 0)
    onehot = (selt == eids).astype(jnp.int32)
    inc = onehot
    k = 1
    while k < T:
        inc = inc + jnp.concatenate(
            [jnp.zeros((E, k), jnp.int32), inc[:, :T - k]], axis=1)
        k *= 2
    rank = inc - onehot
    counts = inc[:, T - 1:T]
    nblk = (counts + BT - 1) // BT
    cum = nblk
    k = 1
    while k < E:
        cum = cum + jnp.concatenate(
            [jnp.zeros((k, 1), jnp.int32), cum[:E - k, :]], axis=0)
        k *= 2
    poff = BT * (cum - nblk)
    pos = jnp.sum(onehot * (poff + rank), axis=0, keepdims=True)
    pos_ref[...] = jnp.broadcast_to(pos, (8, T))
    bids = lax.broadcasted_iota(jnp.int32, (E, NBP), 1)
    be = jnp.sum((jnp.broadcast_to(cum, (E, NBP)) <= bids).astype(jnp.int32),
                 axis=0, keepdims=True)
    be_ref[...] = jnp.broadcast_to(jnp.minimum(be, E - 1), (8, NBP))
    nu_ref[...] = jnp.broadcast_to(cum[E - 1:E, :], (8, 8))


def _route_stage(selt):
    return pl.pallas_call(
        _route_kernel,
        in_specs=[pl.BlockSpec((1, T), lambda: (0, 0))],
        out_specs=[
            pl.BlockSpec((8, T), lambda: (0, 0)),
            pl.BlockSpec((8, NBP), lambda: (0, 0)),
            pl.BlockSpec((8, 8), lambda: (0, 0)),
        ],
        out_shape=[
            jax.ShapeDtypeStruct((8, T), jnp.int32),
            jax.ShapeDtypeStruct((8, NBP), jnp.int32),
            jax.ShapeDtypeStruct((8, 8), jnp.int32),
        ],
        grid=(),
    )(selt)


def _sc_scatter_stage(x2, xr, pos):
    mesh = plsc.VectorSubcoreMesh(core_axis_name="c", subcore_axis_name="s")

    @functools.partial(
        pl.kernel, mesh=mesh,
        out_type=[
            jax.ShapeDtypeStruct((T_PAD, D), jnp.float32),
            jax.ShapeDtypeStruct((T_PAD, D), jnp.float32),
        ],
        scratch_types=[
            pltpu.VMEM((RW,), jnp.int32),
            pltpu.VMEM((RW, D), jnp.float32),
            pltpu.VMEM((RW, D), jnp.float32),
            pltpu.SemaphoreType.DMA,
            pltpu.SemaphoreType.DMA,
        ],
    )
    def k(x2_hbm, xr_hbm, pos_hbm, x2s_hbm, xrs_hbm,
          idx_v, buf_a, buf_b, sem_a, sem_b):
        wid = lax.axis_index("s") * SC_NC + lax.axis_index("c")
        base = wid * RW
        pltpu.sync_copy(pos_hbm.at[pl.ds(base, RW)], idx_v)
        pltpu.sync_copy(x2_hbm.at[pl.ds(base, RW)], buf_a)
        pltpu.sync_copy(xr_hbm.at[pl.ds(base, RW)], buf_b)
        a = pltpu.async_copy(buf_a, x2s_hbm.at[idx_v], sem_a)
        b = pltpu.async_copy(buf_b, xrs_hbm.at[idx_v], sem_b)
        a.wait()
        b.wait()

    return k(x2, xr, pos)


def _ffn_kernel(be_ref, nu_ref, x2s_ref, xrs_ref, w1_ref, b1_ref,
                w2_ref, b2_ref, o_ref):
    i = pl.program_id(0)

    @pl.when(i < nu_ref[0])
    def _():
        x = x2s_ref[...]
        h = lax.dot_general(x, w1_ref[0], DN11,
                            preferred_element_type=jnp.float32)
        h = jnp.maximum(h + b1_ref[0], 0.0)
        y = lax.dot_general(h, w2_ref[0], DN11,
                            preferred_element_type=jnp.float32)
        o_ref[...] = y + b2_ref[0] + xrs_ref[...]


def _ffn_stage(x2s, xrs, W1, b1, W2, b2, blk_expert, nused):
    grid_spec = pltpu.PrefetchScalarGridSpec(
        num_scalar_prefetch=2,
        grid=(NBP,),
        in_specs=[
            pl.BlockSpec((BT, D), lambda i, be, nu: (i, 0)),
            pl.BlockSpec((BT, D), lambda i, be, nu: (i, 0)),
            pl.BlockSpec((1, DFF, D), lambda i, be, nu: (be[i], 0, 0)),
            pl.BlockSpec((1, 1, DFF), lambda i, be, nu: (be[i], 0, 0)),
            pl.BlockSpec((1, D, DFF), lambda i, be, nu: (be[i], 0, 0)),
            pl.BlockSpec((1, 1, D), lambda i, be, nu: (be[i], 0, 0)),
        ],
        out_specs=pl.BlockSpec((BT, D), lambda i, be, nu: (i, 0)),
    )
    return pl.pallas_call(
        _ffn_kernel,
        grid_spec=grid_spec,
        out_shape=jax.ShapeDtypeStruct((T_PAD, D), jnp.float32),
    )(blk_expert, nused, x2s, xrs, W1, b1.reshape(E, 1, DFF), W2,
      b2.reshape(E, 1, D))


def _sc_gather_stage(outs, pos):
    mesh = plsc.VectorSubcoreMesh(core_axis_name="c", subcore_axis_name="s")

    @functools.partial(
        pl.kernel, mesh=mesh,
        out_type=jax.ShapeDtypeStruct((T, D), jnp.float32),
        scratch_types=[
            pltpu.VMEM((RW,), jnp.int32),
            pltpu.VMEM((RW, D), jnp.float32),
            pltpu.SemaphoreType.DMA,
        ],
    )
    def k(outs_hbm, pos_hbm, out_hbm, idx_v, buf, sem):
        wid = lax.axis_index("s") * SC_NC + lax.axis_index("c")
        base = wid * RW
        pltpu.sync_copy(pos_hbm.at[pl.ds(base, RW)], idx_v)
        pltpu.async_copy(outs_hbm.at[idx_v], buf, sem).wait()
        pltpu.sync_copy(buf, out_hbm.at[pl.ds(base, RW)])

    return k(outs, pos)


def kernel(x, ln1_g, ln1_b, Wq, Wk, Wv, Wp, bp, ln2_g, ln2_b,
           Wg, W1, b1, W2, b2):
    Bx = x.shape[0]
    xf = x.reshape(T, D)
    wqk = jnp.concatenate([
        Wq.reshape(H * HS, D),
        Wk.reshape(H * HS, D),
    ], axis=0)
    wv2 = Wv.reshape(H * HS, D)
    qkt, v = _qkv_stage(xf, ln1_g, ln1_b, wqk, wv2)
    att = _attn_stage(qkt, v)
    xr, x2, sel8 = _post_stage(xf, att, Wp, bp, ln2_g, ln2_b, Wg)
    pos8, be8, nu8 = _route_stage(sel8[:1])
    pos = pos8[0]
    x2s, xrs = _sc_scatter_stage(x2, xr, pos)
    outs = _ffn_stage(x2s, xrs, W1, b1, W2, b2,
                      be8[0], nu8[0, :1])
    out = _sc_gather_stage(outs, pos)
    return out.reshape(Bx, T, D)


# speedup vs baseline: 1.7560x; 1.2401x over previous
import functools

import jax
import jax.numpy as jnp
from jax import lax
from jax.experimental import pallas as pl
from jax.experimental.pallas import tpu as pltpu
from jax.experimental.pallas import tpu_sc as plsc

T, D, H, HS, E, DFF = 2048, 768, 12, 64, 8, 3072
BQ = 256
BT = 128
NBP = T // BT + E
T_PAD = NBP * BT

SC_NC, SC_NS = 2, 16
SC_NW = SC_NC * SC_NS
RW = T // SC_NW

DN11 = (((1,), (1,)), ((), ()))
DN00 = (((0,), (0,)), ((), ()))
DN10 = (((1,), (0,)), ((), ()))
DN01 = (((0,), (1,)), ((), ()))


def _halv(acc):
    w = acc.shape[1]
    while w > 1:
        w //= 2
        acc = acc[:, :w] + acc[:, w:]
    return acc


def _xsum(x):
    nc = x.shape[1] // 128
    a = x[:, :128]
    for i in range(1, nc):
        a = a + x[:, i * 128:(i + 1) * 128]
    s = None
    for i in range(0, 128, 8):
        g = a[:, i:i + 8]
        s = g if s is None else s + g
    return _halv(s)


def _subsum(x):
    a = x[:8]
    for i in range(1, x.shape[0] // 8):
        a = a + x[i * 8:(i + 1) * 8]
    a = a[:4] + a[4:]
    a = a[:2] + a[2:]
    return a[:1] + a[1:]


def _ln_rows(x, g, b):
    n = x.shape[1]
    m = _xsum(x) / n
    v = _xsum((x - m) ** 2) / n
    return (x - m) / jnp.sqrt(v + 1e-5) * g + b


def _qkv_kernel(x_ref, g_ref, b_ref, wqk_ref, wv_ref, qkt_ref, v_ref):
    x1 = _ln_rows(x_ref[...], g_ref[...], b_ref[...])
    qkt = lax.dot_general(wqk_ref[...], x1, DN11,
                          preferred_element_type=jnp.float32)
    qkt_ref[...] = qkt.astype(jnp.bfloat16)
    v_ref[...] = lax.dot_general(x1, wv_ref[...], DN11,
                                 preferred_element_type=jnp.float32)


def _qkv_stage(xf, ln1_g, ln1_b, wqk, wv):
    return pl.pallas_call(
        _qkv_kernel,
        grid=(T // BQ,),
        in_specs=[
            pl.BlockSpec((BQ, D), lambda i: (i, 0)),
            pl.BlockSpec((1, D), lambda i: (0, 0)),
            pl.BlockSpec((1, D), lambda i: (0, 0)),
            pl.BlockSpec((2 * D, D), lambda i: (0, 0)),
            pl.BlockSpec((D, D), lambda i: (0, 0)),
        ],
        out_specs=[
            pl.BlockSpec((2 * D, BQ), lambda i: (0, i)),
            pl.BlockSpec((BQ, D), lambda i: (i, 0)),
        ],
        out_shape=[
            jax.ShapeDtypeStruct((2 * D, T), jnp.bfloat16),
            jax.ShapeDtypeStruct((T, D), jnp.float32),
        ],
    )(xf, ln1_g.reshape(1, D), ln1_b.reshape(1, D), wqk, wv)


def _attn_kernel(qt_ref, kt_ref, v_ref, o_ref):
    i = pl.program_id(0)
    scale = jnp.float32(D ** -0.5)
    rows = lax.broadcasted_iota(jnp.int32, (BQ, T), 0) + i * BQ
    cols = lax.broadcasted_iota(jnp.int32, (BQ, T), 1)
    CK = 1024
    outs = []
    for h in range(H):
        qth = qt_ref[h * HS:(h + 1) * HS, :]
        kth = kt_ref[D + h * HS: D + (h + 1) * HS, :]
        vh = v_ref[:, h * HS:(h + 1) * HS]
        s = lax.dot_general(qth, kth, DN00,
                            preferred_element_type=jnp.float32) * scale
        s = jnp.where(cols <= rows, s, -jnp.inf)
        s0 = s[:, :CK]
        m0 = jnp.max(s0, axis=1, keepdims=True)
        p0 = jnp.exp(s0 - m0)
        l0 = jnp.sum(p0, axis=1, keepdims=True)
        acc0 = lax.dot_general(p0, vh[:CK], DN10,
                               preferred_element_type=jnp.float32)
        att0 = acc0 * (jnp.float32(1.0) / l0)
        s1 = s[:, CK:]
        m1c = jnp.max(s1, axis=1, keepdims=True)
        m1 = jnp.maximum(m0, m1c)
        delta = jnp.where(m0 == m1, jnp.float32(0.0), m0 - m1)
        p1 = jnp.exp(s1 - m1)
        l1c = jnp.sum(p1, axis=1, keepdims=True)
        ed = jnp.exp(delta)
        l1 = ed * l0 + l1c
        carry = (ed * l0) * att0
        acc1 = lax.dot_general(p1, vh[CK:], DN10,
                               preferred_element_type=jnp.float32) + carry
        outs.append(acc1 * (jnp.float32(1.0) / l1))
    att = jnp.concatenate(outs, axis=1)
    o_ref[...] = att.T.astype(jnp.bfloat16)


def _attn_stage(qkt, v):
    return pl.pallas_call(
        _attn_kernel,
        grid=(T // BQ,),
        in_specs=[
            pl.BlockSpec((D, BQ), lambda i: (0, i)),
            pl.BlockSpec((2 * D, T), lambda i: (0, 0)),
            pl.BlockSpec((T, D), lambda i: (0, 0)),
        ],
        out_specs=pl.BlockSpec((D, BQ), lambda i: (0, i)),
        out_shape=jax.ShapeDtypeStruct((D, T), jnp.bfloat16),
    )(qkt, qkt, v)


def _post_kernel(x_ref, at_ref, wp_ref, bp_ref, g2_ref, b2_ref, wg_ref,
                 xr_ref, x2_ref, sel_ref):
    xt = x_ref[...].T
    projt = lax.dot_general(wp_ref[...], at_ref[...], DN10,
                            preferred_element_type=jnp.float32)
    xrt = xt + (projt + bp_ref[...])
    n = xrt.shape[0]
    m = _subsum(xrt) / n
    var = _subsum((xrt - m) ** 2) / n
    x2t = (xrt - m) / jnp.sqrt(var + 1e-5) * g2_ref[...] + b2_ref[...]
    glt = lax.dot_general(wg_ref[...], x2t, DN10,
                          preferred_element_type=jnp.float32)
    mx = jnp.max(glt, axis=0, keepdims=True)
    eidx = lax.broadcasted_iota(jnp.int32, (E, BQ), 0)
    sel = jnp.min(jnp.where(glt == mx, eidx, E), axis=0, keepdims=True)
    xr_ref[...] = xrt.T
    x2_ref[...] = x2t.T
    sel_ref[...] = jnp.broadcast_to(sel, (8, BQ))


def _post_stage(xf, att, Wp, bp, ln2_g, ln2_b, Wg):
    return pl.pallas_call(
        _post_kernel,
        grid=(T // BQ,),
        in_specs=[
            pl.BlockSpec((BQ, D), lambda i: (i, 0)),
            pl.BlockSpec((D, BQ), lambda i: (0, i)),
            pl.BlockSpec((D, D), lambda i: (0, 0)),
            pl.BlockSpec((D, 1), lambda i: (0, 0)),
            pl.BlockSpec((D, 1), lambda i: (0, 0)),
            pl.BlockSpec((D, 1), lambda i: (0, 0)),
            pl.BlockSpec((E, D), lambda i: (0, 0)),
        ],
        out_specs=[
            pl.BlockSpec((BQ, D), lambda i: (i, 0)),
            pl.BlockSpec((BQ, D), lambda i: (i, 0)),
            pl.BlockSpec((8, BQ), lambda i: (0, i)),
        ],
        out_shape=[
            jax.ShapeDtypeStruct((T, D), jnp.float32),
            jax.ShapeDtypeStruct((T, D), jnp.float32),
            jax.ShapeDtypeStruct((8, T), jnp.int32),
        ],
    )(xf, att, Wp, bp.reshape(D, 1), ln2_g.reshape(D, 1),
      ln2_b.reshape(D, 1), Wg)


def _route_kernel(sel_ref, pos_ref, be_ref, nu_ref):
    selt = sel_ref[...]
    eids = lax.broadcasted_iota(jnp.int32, (E, T), 0)
    onehot = (selt == eids).astype(jnp.int32)
    inc = onehot
    k = 1
    while k < T:
        inc = inc + jnp.concatenate(
            [jnp.zeros((E, k), jnp.int32), inc[:, :T - k]], axis=1)
        k *= 2
    rank = inc - onehot
    counts = inc[:, T - 1:T]
    nblk = (counts + BT - 1) // BT
    cum = nblk
    k = 1
    while k < E:
        cum = cum + jnp.concatenate(
            [jnp.zeros((k, 1), jnp.int32), cum[:E - k, :]], axis=0)
        k *= 2
    poff = BT * (cum - nblk)
    pos = jnp.sum(onehot * (poff + rank), axis=0, keepdims=True)
    pos_ref[...] = jnp.broadcast_to(pos, (8, T))
    bids = lax.broadcasted_iota(jnp.int32, (E, NBP), 1)
    be = jnp.sum((jnp.broadcast_to(cum, (E, NBP)) <= bids).astype(jnp.int32),
                 axis=0, keepdims=True)
    be_ref[...] = jnp.broadcast_to(jnp.minimum(be, E - 1), (8, NBP))
    nu_ref[...] = jnp.broadcast_to(cum[E - 1:E, :], (8, 8))


def _route_stage(selt):
    return pl.pallas_call(
        _route_kernel,
        in_specs=[pl.BlockSpec((1, T), lambda: (0, 0))],
        out_specs=[
            pl.BlockSpec((8, T), lambda: (0, 0)),
            pl.BlockSpec((8, NBP), lambda: (0, 0)),
            pl.BlockSpec((8, 8), lambda: (0, 0)),
        ],
        out_shape=[
            jax.ShapeDtypeStruct((8, T), jnp.int32),
            jax.ShapeDtypeStruct((8, NBP), jnp.int32),
            jax.ShapeDtypeStruct((8, 8), jnp.int32),
        ],
        grid=(),
    )(selt)


def _sc_scatter_stage(x2, xr, pos):
    mesh = plsc.VectorSubcoreMesh(core_axis_name="c", subcore_axis_name="s")

    @functools.partial(
        pl.kernel, mesh=mesh,
        out_type=[
            jax.ShapeDtypeStruct((T_PAD, D), jnp.float32),
            jax.ShapeDtypeStruct((T_PAD, D), jnp.float32),
        ],
        scratch_types=[
            pltpu.VMEM((RW,), jnp.int32),
            pltpu.VMEM((RW, D), jnp.float32),
            pltpu.VMEM((RW, D), jnp.float32),
            pltpu.SemaphoreType.DMA,
            pltpu.SemaphoreType.DMA,
        ],
    )
    def k(x2_hbm, xr_hbm, pos_hbm, x2s_hbm, xrs_hbm,
          idx_v, buf_a, buf_b, sem_a, sem_b):
        wid = lax.axis_index("s") * SC_NC + lax.axis_index("c")
        base = wid * RW
        pltpu.sync_copy(pos_hbm.at[pl.ds(base, RW)], idx_v)
        pltpu.sync_copy(x2_hbm.at[pl.ds(base, RW)], buf_a)
        pltpu.sync_copy(xr_hbm.at[pl.ds(base, RW)], buf_b)
        a = pltpu.async_copy(buf_a, x2s_hbm.at[idx_v], sem_a)
        b = pltpu.async_copy(buf_b, xrs_hbm.at[idx_v], sem_b)
        a.wait()
        b.wait()

    return k(x2, xr, pos)


def _ffn_kernel(be_ref, nu_ref, x2s_ref, xrs_ref, w1_ref, b1_ref,
                w2_ref, b2_ref, o_ref):
    i = pl.program_id(0)

    @pl.when(i < nu_ref[0])
    def _():
        x = x2s_ref[...]
        h = lax.dot_general(x, w1_ref[0], DN11,
                            preferred_element_type=jnp.float32)
        h = jnp.maximum(h + b1_ref[0], 0.0)
        y = lax.dot_general(h, w2_ref[0], DN11,
                            preferred_element_type=jnp.float32)
        o_ref[...] = y + b2_ref[0] + xrs_ref[...]


def _ffn_stage(x2s, xrs, W1, b1, W2, b2, blk_expert, nused):
    grid_spec = pltpu.PrefetchScalarGridSpec(
        num_scalar_prefetch=2,
        grid=(NBP,),
        in_specs=[
            pl.BlockSpec((BT, D), lambda i, be, nu: (i, 0)),
            pl.BlockSpec((BT, D), lambda i, be, nu: (i, 0)),
            pl.BlockSpec((1, DFF, D), lambda i, be, nu: (be[i], 0, 0)),
            pl.BlockSpec((1, 1, DFF), lambda i, be, nu: (be[i], 0, 0)),
            pl.BlockSpec((1, D, DFF), lambda i, be, nu: (be[i], 0, 0)),
            pl.BlockSpec((1, 1, D), lambda i, be, nu: (be[i], 0, 0)),
        ],
        out_specs=pl.BlockSpec((BT, D), lambda i, be, nu: (i, 0)),
    )
    return pl.pallas_call(
        _ffn_kernel,
        grid_spec=grid_spec,
        out_shape=jax.ShapeDtypeStruct((T_PAD, D), jnp.float32),
    )(blk_expert, nused, x2s, xrs, W1, b1.reshape(E, 1, DFF), W2,
      b2.reshape(E, 1, D))


def _sc_gather_stage(outs, pos):
    mesh = plsc.VectorSubcoreMesh(core_axis_name="c", subcore_axis_name="s")

    @functools.partial(
        pl.kernel, mesh=mesh,
        out_type=jax.ShapeDtypeStruct((T, D), jnp.float32),
        scratch_types=[
            pltpu.VMEM((RW,), jnp.int32),
            pltpu.VMEM((RW, D), jnp.float32),
            pltpu.SemaphoreType.DMA,
        ],
    )
    def k(outs_hbm, pos_hbm, out_hbm, idx_v, buf, sem):
        wid = lax.axis_index("s") * SC_NC + lax.axis_index("c")
        base = wid * RW
        pltpu.sync_copy(pos_hbm.at[pl.ds(base, RW)], idx_v)
        pltpu.async_copy(outs_hbm.at[idx_v], buf, sem).wait()
        pltpu.sync_copy(buf, out_hbm.at[pl.ds(base, RW)])

    return k(outs, pos)


def kernel(x, ln1_g, ln1_b, Wq, Wk, Wv, Wp, bp, ln2_g, ln2_b,
           Wg, W1, b1, W2, b2):
    Bx = x.shape[0]
    xf = x.reshape(T, D)
    wqk = jnp.concatenate([
        Wq.reshape(H * HS, D),
        Wk.reshape(H * HS, D),
    ], axis=0)
    wv2 = Wv.reshape(H * HS, D)
    qkt, v = _qkv_stage(xf, ln1_g, ln1_b, wqk, wv2)
    att = _attn_stage(qkt, v)
    xr, x2, sel8 = _post_stage(xf, att, Wp, bp, ln2_g, ln2_b, Wg)
    pos8, be8, nu8 = _route_stage(sel8[:1])
    pos = pos8[0]
    x2s, xrs = _sc_scatter_stage(x2, xr, pos)
    outs = _ffn_stage(x2s, xrs, W1, b1, W2, b2,
                      be8[0], nu8[0, :1])
    out = _sc_gather_stage(outs, pos)
    return out.reshape(Bx, T, D)
